# baseline (speedup 1.0000x reference)
import numpy as np

import jax
import jax.numpy as jnp
from jax.experimental import pallas as pl
from jax.experimental.pallas import tpu as pltpu

_NH = 16
_NW = 16
_N_ITERS = 5
_C = 192
_H = 224
_W = 224
_BLK = 14
_RPS = 2
_L = _RPS * _BLK * _W
_WIN = (_RPS + 2) * _NW
_P = _H * _W
_K = _NH * _NW
_NSTEP = (_H // _BLK) // (2 * _RPS)
_IL = 2 * _L
_NEG = -1e30


def _build_masks():
    q = np.arange(_L)
    sr = q // (_BLK * _W)
    cb = (q % _W) // _BLK
    w = np.arange(_WIN)
    wr = w // _NW
    wc = w % _NW
    col_ok = np.abs(wc[:, None] - cb[None, :]) <= 1
    row_ok = np.abs(wr[:, None] - 1 - sr[None, :]) <= 1
    base = col_ok & row_ok
    top = base & (wr[:, None] != 0)
    bot = base & (wr[:, None] != _RPS + 1)
    mask = np.stack([
        np.where(top, 0.0, _NEG),
        np.where(base, 0.0, _NEG),
        np.where(bot, 0.0, _NEG),
    ]).astype(np.float32)

    cell = sr * _NW + cb
    sel = (np.arange(_RPS * _NW)[:, None] == cell[None, :])
    return mask, sel.astype(np.float32)


_MASK_NP, _SEL_NP = _build_masks()


def _ssn_body(pix_ref, mask_a, mask_b, sel_ref, spf_ref, lab_ref,
              cent, accn, accd, pxbf):
    it = pl.program_id(0)
    gg = pl.program_id(1)

    @pl.when(jnp.logical_and(it == 0, gg == 0))
    def _():
        accn[:, :] = jnp.zeros_like(accn)
        accd[:, :] = jnp.zeros_like(accd)

    @pl.when(it == 0)
    def _():
        flat = pix_ref[:, :, :].reshape(_C, _IL)
        pxbf[0:_C, pl.ds(_IL * gg, _IL)] = flat.astype(jnp.bfloat16)
        pxbf[_C:_C + 2, pl.ds(_IL * gg, _IL)] = jnp.ones(
            (2, _IL), jnp.bfloat16)
        sel = sel_ref[:, :]
        sums_a = jax.lax.dot_general(
            sel, flat[:, :_L], (((1,), (1,)), ((), ())),
            preferred_element_type=jnp.float32)
        sums_b = jax.lax.dot_general(
            sel, flat[:, _L:], (((1,), (1,)), ((), ())),
            preferred_element_type=jnp.float32)
        base = _NW * (2 * _RPS * gg + 1)
        accn[pl.ds(base, _RPS * _NW), :] = sums_a
        accn[pl.ds(base + _RPS * _NW, _RPS * _NW), :] = sums_b
        accd[pl.ds(base, 2 * _RPS * _NW), :] = jnp.full(
            (2 * _RPS * _NW, 1), float(_BLK * _BLK), jnp.float32)

    @pl.when(jnp.logical_and(it > 0, gg == 0))
    def _():
        cent[:, :] = accn[:, :] / (accd[:, :] + 1e-16)
        accn[:, :] = jnp.zeros_like(accn)
        accd[:, :] = jnp.zeros_like(accd)

    def _sub_slab(g, mask):
        px = pxbf[:, pl.ds(_L * g, _L)]
        cw = cent[pl.ds(_NW * _RPS * g, _WIN), :]
        nss = -jnp.sum(cw * cw, axis=1, keepdims=True)
        ns_hi = nss.astype(jnp.bfloat16)
        ns_lo = (nss - ns_hi.astype(jnp.float32)).astype(jnp.bfloat16)
        cw_aug = jnp.concatenate(
            [(cw + cw).astype(jnp.bfloat16), ns_hi, ns_lo], axis=1)
        logits = jax.lax.dot_general(
            cw_aug, px, (((1,), (0,)), ((), ())),
            preferred_element_type=jnp.float32) + mask
        m = jnp.max(logits, axis=0, keepdims=True)
        e = jnp.exp(logits - m)
        a = e / jnp.sum(e, axis=0, keepdims=True)
        contrib = jax.lax.dot_general(
            a.astype(jnp.bfloat16), px, (((1,), (1,)), ((), ())),
            preferred_element_type=jnp.float32)
        base = _NW * _RPS * g
        accn[pl.ds(base, _WIN), :] += contrib[:, :_C]
        accd[pl.ds(base, _WIN), :] += contrib[:, _C:_C + 1]

        @pl.when(it == _N_ITERS)
        def _():
            wi = jax.lax.broadcasted_iota(jnp.int32, (_WIN, _L), 0)
            cand = jnp.where(logits >= m, wi, _WIN)
            lw = jnp.min(cand, axis=0)
            k = _NW * (_RPS * g - 1) + lw
            lab_ref[:, pl.ds(_L * g, _L)] = k.reshape(1, _L)

    @pl.when(it > 0)
    def _():
        _sub_slab(2 * gg, mask_a[0])
        _sub_slab(2 * gg + 1, mask_b[0])

    @pl.when(jnp.logical_and(it == _N_ITERS, gg == _NSTEP - 1))
    def _():
        spf_ref[0, :, :] = accn[_NW:_NW + _K, :] / (accd[_NW:_NW + _K, :] +
                                                    1e-16)


def kernel(f):
    pix = f.reshape(_C, _H, _W)
    mask = jnp.asarray(_MASK_NP)
    sel = jnp.asarray(_SEL_NP)
    spf, lab = pl.pallas_call(
        _ssn_body,
        grid=(_N_ITERS + 1, _NSTEP),
        in_specs=[
            pl.BlockSpec(
                (_C, _IL // _W, _W),
                lambda it, gg: (0, jnp.where(it == 0, gg, 0), 0)),
            pl.BlockSpec(
                (1, _WIN, _L),
                lambda it, gg: (jnp.where(gg == 0, 0, 1), 0, 0)),
            pl.BlockSpec(
                (1, _WIN, _L),
                lambda it, gg: (jnp.where(gg == _NSTEP - 1, 2, 1), 0, 0)),
            pl.BlockSpec((_RPS * _NW, _L), lambda it, gg: (0, 0)),
        ],
        out_specs=[
            pl.BlockSpec((1, _K, _C), lambda it, gg: (0, 0, 0)),
            pl.BlockSpec((1, _P), lambda it, gg: (0, 0)),
        ],
        out_shape=[
            jax.ShapeDtypeStruct((1, _K, _C), jnp.float32),
            jax.ShapeDtypeStruct((1, _P), jnp.int32),
        ],
        scratch_shapes=[
            pltpu.VMEM(((_NH + 2) * _NW, _C), jnp.float32),
            pltpu.VMEM(((_NH + 2) * _NW, _C), jnp.float32),
            pltpu.VMEM(((_NH + 2) * _NW, 1), jnp.float32),
            pltpu.VMEM((_C + 2, _P), jnp.bfloat16),
        ],
    )(pix, mask, mask, sel)
    return spf, lab


# speedup vs baseline: 1.1526x; 1.1526x over previous
import numpy as np

import jax
import jax.numpy as jnp
from jax.experimental import pallas as pl
from jax.experimental.pallas import tpu as pltpu

_NH = 16
_NW = 16
_N_ITERS = 5
_C = 192
_H = 224
_W = 224
_BLK = 14
_RPS = 2
_L = _RPS * _BLK * _W
_WIN = (_RPS + 2) * _NW
_P = _H * _W
_K = _NH * _NW
_NSLAB = (_H // _BLK) // _RPS
_IRPS = 4
_IL = _IRPS * _BLK * _W
_NEG = -1e30


def _build_masks():
    q = np.arange(_L)
    sr = q // (_BLK * _W)
    cb = (q % _W) // _BLK
    w = np.arange(_WIN)
    wr = w // _NW
    wc = w % _NW
    col_ok = np.abs(wc[:, None] - cb[None, :]) <= 1
    row_ok = np.abs(wr[:, None] - 1 - sr[None, :]) <= 1
    base = col_ok & row_ok
    top = base & (wr[:, None] != 0)
    bot = base & (wr[:, None] != _RPS + 1)
    mask = np.stack([
        np.where(top, 0.0, _NEG),
        np.where(base, 0.0, _NEG),
        np.where(bot, 0.0, _NEG),
    ]).astype(np.float32)

    cell = sr * _NW + cb
    sel = (np.arange(_RPS * _NW)[:, None] == cell[None, :])
    return mask, sel.astype(np.float32)


_MASK_NP, _SEL_NP = _build_masks()


def _ssn_body(pix_ref, mask_ref, sel_ref, spf_ref, lab_ref, cent, accn, accd,
              pxbf):
    it = pl.program_id(0)
    g = pl.program_id(1)

    @pl.when(jnp.logical_and(it == 0, g == 0))
    def _():
        accn[:, :] = jnp.zeros_like(accn)
        accd[:, :] = jnp.zeros_like(accd)

    @pl.when(jnp.logical_and(it == 0, g < _NSLAB // 2))
    def _():
        flat = pix_ref[:, :, :].reshape(_C, _IL)
        pxbf[0:_C, pl.ds(_IL * g, _IL)] = flat.astype(jnp.bfloat16)
        pxbf[_C:_C + 2, pl.ds(_IL * g, _IL)] = jnp.ones(
            (2, _IL), jnp.bfloat16)
        sel = sel_ref[:, :]
        sums_a = jax.lax.dot_general(
            sel, flat[:, :_L], (((1,), (1,)), ((), ())),
            preferred_element_type=jnp.float32)
        sums_b = jax.lax.dot_general(
            sel, flat[:, _L:], (((1,), (1,)), ((), ())),
            preferred_element_type=jnp.float32)
        base = _NW * (_IRPS * g + 1)
        accn[pl.ds(base, _RPS * _NW), :] = sums_a
        accn[pl.ds(base + _RPS * _NW, _RPS * _NW), :] = sums_b
        accd[pl.ds(base, _IRPS * _NW), :] = jnp.full(
            (_IRPS * _NW, 1), float(_BLK * _BLK), jnp.float32)

    @pl.when(jnp.logical_and(it > 0, g == 0))
    def _():
        cent[:, :] = accn[:, :] / (accd[:, :] + 1e-16)
        accn[:, :] = jnp.zeros_like(accn)
        accd[:, :] = jnp.zeros_like(accd)

    @pl.when(it > 0)
    def _():
        px = pxbf[:, pl.ds(_L * g, _L)]
        cw = cent[pl.ds(_NW * _RPS * g, _WIN), :]
        nss = -jnp.sum(cw * cw, axis=1, keepdims=True)
        ns_hi = nss.astype(jnp.bfloat16)
        ns_lo = (nss - ns_hi.astype(jnp.float32)).astype(jnp.bfloat16)
        cw_aug = jnp.concatenate(
            [(cw + cw).astype(jnp.bfloat16), ns_hi, ns_lo], axis=1)
        logits = jax.lax.dot_general(
            cw_aug, px, (((1,), (0,)), ((), ())),
            preferred_element_type=jnp.float32) + mask_ref[0]
        m = jnp.max(logits, axis=0, keepdims=True)
        e = jnp.exp(logits - m)
        a = e / jnp.sum(e, axis=0, keepdims=True)
        contrib = jax.lax.dot_general(
            a.astype(jnp.bfloat16), px, (((1,), (1,)), ((), ())),
            preferred_element_type=jnp.float32)
        base = _NW * _RPS * g
        accn[pl.ds(base, _WIN), :] += contrib[:, :_C]
        accd[pl.ds(base, _WIN), :] += contrib[:, _C:_C + 1]

        @pl.when(it == _N_ITERS)
        def _():
            wi = jax.lax.broadcasted_iota(jnp.int32, (_WIN, _L), 0)
            cand = jnp.where(logits >= m, wi, _WIN)
            lw = jnp.min(cand, axis=0)
            k = _NW * (_RPS * g - 1) + lw
            lab_ref[:, pl.ds(_L * g, _L)] = k.reshape(1, _L)

    @pl.when(jnp.logical_and(it == _N_ITERS, g == _NSLAB - 1))
    def _():
        spf_ref[0, :, :] = accn[_NW:_NW + _K, :] / (accd[_NW:_NW + _K, :] +
                                                    1e-16)


def kernel(f):
    pix = f.reshape(_C, _H, _W)
    mask = jnp.asarray(_MASK_NP)
    sel = jnp.asarray(_SEL_NP)
    spf, lab = pl.pallas_call(
        _ssn_body,
        grid=(_N_ITERS + 1, _NSLAB),
        in_specs=[
            pl.BlockSpec(
                (_C, _IRPS * _BLK, _W),
                lambda it, g: (0,
                               jnp.where(it == 0,
                                         jnp.minimum(g, _NSLAB // 2 - 1), 0),
                               0)),
            pl.BlockSpec(
                (1, _WIN, _L),
                lambda it, g: (jnp.where(g == 0, 0,
                                         jnp.where(g == _NSLAB - 1, 2, 1)),
                               0, 0)),
            pl.BlockSpec((_RPS * _NW, _L), lambda it, g: (0, 0)),
        ],
        out_specs=[
            pl.BlockSpec((1, _K, _C), lambda it, g: (0, 0, 0)),
            pl.BlockSpec((1, _P), lambda it, g: (0, 0)),
        ],
        out_shape=[
            jax.ShapeDtypeStruct((1, _K, _C), jnp.float32),
            jax.ShapeDtypeStruct((1, _P), jnp.int32),
        ],
        scratch_shapes=[
            pltpu.VMEM(((_NH + 2) * _NW, _C), jnp.float32),
            pltpu.VMEM(((_NH + 2) * _NW, _C), jnp.float32),
            pltpu.VMEM(((_NH + 2) * _NW, 1), jnp.float32),
            pltpu.VMEM((_C + 2, _P), jnp.bfloat16),
        ],
    )(pix, mask, sel)
    return spf, lab


# speedup vs baseline: 1.1541x; 1.0013x over previous
import numpy as np

import jax
import jax.numpy as jnp
from jax.experimental import pallas as pl
from jax.experimental.pallas import tpu as pltpu

_NH = 16
_NW = 16
_N_ITERS = 5
_C = 192
_H = 224
_W = 224
_BLK = 14
_RPS = 2
_L = _RPS * _BLK * _W
_WIN = (_RPS + 2) * _NW
_P = _H * _W
_K = _NH * _NW
_NSLAB = (_H // _BLK) // _RPS
_IRPS = 4
_IL = _IRPS * _BLK * _W
_NEG = -1e30


def _build_masks():
    q = np.arange(_L)
    sr = q // (_BLK * _W)
    cb = (q % _W) // _BLK
    w = np.arange(_WIN)
    wr = w // _NW
    wc = w % _NW
    col_ok = np.abs(wc[:, None] - cb[None, :]) <= 1
    row_ok = np.abs(wr[:, None] - 1 - sr[None, :]) <= 1
    base = col_ok & row_ok
    top = base & (wr[:, None] != 0)
    bot = base & (wr[:, None] != _RPS + 1)
    mask = np.stack([
        np.where(top, 0.0, _NEG),
        np.where(base, 0.0, _NEG),
        np.where(bot, 0.0, _NEG),
    ]).astype(np.float32)

    cell = sr * _NW + cb
    sel = (np.arange(_RPS * _NW)[:, None] == cell[None, :])
    return mask, sel.astype(np.float32)


_MASK_NP, _SEL_NP = _build_masks()


def _ssn_body(pix_ref, mask_ref, sel_ref, spf_ref, lab_ref, cent, accn, accd,
              pxbf):
    it = pl.program_id(0)
    g = pl.program_id(1)

    @pl.when(jnp.logical_and(it == 0, g == 0))
    def _():
        accn[:, :] = jnp.zeros_like(accn)
        accd[:, :] = jnp.zeros_like(accd)

    @pl.when(jnp.logical_and(it == 0, g < _NSLAB // 2))
    def _():
        flat = pix_ref[:, :, :].reshape(_C, _IL)
        pxbf[0:_C, pl.ds(_IL * g, _IL)] = flat.astype(jnp.bfloat16)
        pxbf[_C:_C + 2, pl.ds(_IL * g, _IL)] = jnp.ones(
            (2, _IL), jnp.bfloat16)
        sel = sel_ref[:, :]
        sums_a = jax.lax.dot_general(
            sel, flat[:, :_L], (((1,), (1,)), ((), ())),
            preferred_element_type=jnp.float32)
        sums_b = jax.lax.dot_general(
            sel, flat[:, _L:], (((1,), (1,)), ((), ())),
            preferred_element_type=jnp.float32)
        base = _NW * (_IRPS * g + 1)
        accn[pl.ds(base, _RPS * _NW), :] = sums_a
        accn[pl.ds(base + _RPS * _NW, _RPS * _NW), :] = sums_b
        accd[pl.ds(base, _IRPS * _NW), :] = jnp.full(
            (_IRPS * _NW, 1), float(_BLK * _BLK), jnp.float32)

    @pl.when(jnp.logical_and(it > 0, g == 0))
    def _():
        cent[:, :] = accn[:, :] / (accd[:, :] + 1e-16)
        accn[:, :] = jnp.zeros_like(accn)
        accd[:, :] = jnp.zeros_like(accd)

    @pl.when(it > 0)
    def _():
        px = pxbf[:, pl.ds(_L * g, _L)]
        cw = cent[pl.ds(_NW * _RPS * g, _WIN), :]
        nss = -jnp.sum(cw * cw, axis=1, keepdims=True)
        ns_hi = nss.astype(jnp.bfloat16)
        ns_lo = (nss - ns_hi.astype(jnp.float32)).astype(jnp.bfloat16)
        cw_aug = jnp.concatenate(
            [(cw + cw).astype(jnp.bfloat16), ns_hi, ns_lo], axis=1)
        logits = jax.lax.dot_general(
            cw_aug, px, (((1,), (0,)), ((), ())),
            preferred_element_type=jnp.float32) + mask_ref[0]
        m = jnp.max(logits, axis=0, keepdims=True)
        e = jnp.exp(logits - m)
        zr = 1.0 / jnp.sum(e, axis=0, keepdims=True)
        a = e * zr
        contrib = jax.lax.dot_general(
            a.astype(jnp.bfloat16), px, (((1,), (1,)), ((), ())),
            preferred_element_type=jnp.float32)
        base = _NW * _RPS * g
        accn[pl.ds(base, _WIN), :] += contrib[:, :_C]
        accd[pl.ds(base, _WIN), :] += contrib[:, _C:_C + 1]

        @pl.when(it == _N_ITERS)
        def _():
            wi = jax.lax.broadcasted_iota(jnp.int32, (_WIN, _L), 0)
            cand = jnp.where(logits >= m, wi, _WIN)
            lw = jnp.min(cand, axis=0)
            k = _NW * (_RPS * g - 1) + lw
            lab_ref[:, pl.ds(_L * g, _L)] = k.reshape(1, _L)

    @pl.when(jnp.logical_and(it == _N_ITERS, g == _NSLAB - 1))
    def _():
        spf_ref[0, :, :] = accn[_NW:_NW + _K, :] / (accd[_NW:_NW + _K, :] +
                                                    1e-16)


def kernel(f):
    pix = f.reshape(_C, _H, _W)
    mask = jnp.asarray(_MASK_NP)
    sel = jnp.asarray(_SEL_NP)
    spf, lab = pl.pallas_call(
        _ssn_body,
        grid=(_N_ITERS + 1, _NSLAB),
        in_specs=[
            pl.BlockSpec(
                (_C, _IRPS * _BLK, _W),
                lambda it, g: (0,
                               jnp.where(it == 0,
                                         jnp.minimum(g, _NSLAB // 2 - 1), 0),
                               0)),
            pl.BlockSpec(
                (1, _WIN, _L),
                lambda it, g: (jnp.where(g == 0, 0,
                                         jnp.where(g == _NSLAB - 1, 2, 1)),
                               0, 0)),
            pl.BlockSpec((_RPS * _NW, _L), lambda it, g: (0, 0)),
        ],
        out_specs=[
            pl.BlockSpec((1, _K, _C), lambda it, g: (0, 0, 0)),
            pl.BlockSpec((1, _P), lambda it, g: (0, 0)),
        ],
        out_shape=[
            jax.ShapeDtypeStruct((1, _K, _C), jnp.float32),
            jax.ShapeDtypeStruct((1, _P), jnp.int32),
        ],
        scratch_shapes=[
            pltpu.VMEM(((_NH + 2) * _NW, _C), jnp.float32),
            pltpu.VMEM(((_NH + 2) * _NW, _C), jnp.float32),
            pltpu.VMEM(((_NH + 2) * _NW, 1), jnp.float32),
            pltpu.VMEM((_C + 2, _P), jnp.bfloat16),
        ],
    )(pix, mask, sel)
    return spf, lab


# speedup vs baseline: 1.2479x; 1.0813x over previous
import numpy as np

import jax
import jax.numpy as jnp
from jax.experimental import pallas as pl
from jax.experimental.pallas import tpu as pltpu

_NH = 16
_NW = 16
_N_ITERS = 5
_C = 192
_H = 224
_W = 224
_BLK = 14
_RPS = 2
_L = _RPS * _BLK * _W
_WIN = (_RPS + 2) * _NW
_P = _H * _W
_K = _NH * _NW
_NSLAB = (_H // _BLK) // _RPS
_NINIT = 4
_IL = _P // _NINIT
_NEG = -1e30


def _build_masks():
    q = np.arange(_L)
    sr = q // (_BLK * _W)
    cb = (q % _W) // _BLK
    w = np.arange(_WIN)
    wr = w // _NW
    wc = w % _NW
    col_ok = np.abs(wc[:, None] - cb[None, :]) <= 1
    row_ok = np.abs(wr[:, None] - 1 - sr[None, :]) <= 1
    base = col_ok & row_ok
    top = base & (wr[:, None] != 0)
    bot = base & (wr[:, None] != _RPS + 1)
    mask = np.stack([
        np.where(top, 0.0, _NEG),
        np.where(base, 0.0, _NEG),
        np.where(bot, 0.0, _NEG),
    ]).astype(np.float32)

    cell = sr * _NW + cb
    sel = (np.arange(_RPS * _NW)[:, None] == cell[None, :])
    return mask, sel.astype(np.float32)


_MASK_NP, _SEL_NP = _build_masks()


def _ssn_body(pix_ref, mask_ref, sel_ref, spf_ref, lab_ref, cent, accn, accd,
              pxbf):
    s = pl.program_id(0)

    @pl.when(s == 0)
    def _():
        accn[:, :] = jnp.zeros_like(accn)
        accd[:, :] = jnp.zeros_like(accd)

    @pl.when(s < _NINIT)
    def _():
        flat = pix_ref[:, :, :].reshape(_C, _IL)
        pxbf[0:_C, pl.ds(_IL * s, _IL)] = flat.astype(jnp.bfloat16)
        pxbf[_C:_C + 2, pl.ds(_IL * s, _IL)] = jnp.ones(
            (2, _IL), jnp.bfloat16)
        sel = sel_ref[:, :]
        sums_a = jax.lax.dot_general(
            sel, flat[:, :_L], (((1,), (1,)), ((), ())),
            preferred_element_type=jnp.float32)
        sums_b = jax.lax.dot_general(
            sel, flat[:, _L:], (((1,), (1,)), ((), ())),
            preferred_element_type=jnp.float32)
        base = _NW * ((_IL // (_BLK * _W)) * s + 1)
        accn[pl.ds(base, _RPS * _NW), :] = sums_a
        accn[pl.ds(base + _RPS * _NW, _RPS * _NW), :] = sums_b
        accd[pl.ds(base, 2 * _RPS * _NW), :] = jnp.full(
            (2 * _RPS * _NW, 1), float(_BLK * _BLK), jnp.float32)

    @pl.when(s >= _NINIT)
    def _():
        cent[:, :] = accn[:, :] / (accd[:, :] + 1e-16)
        accn[:, :] = jnp.zeros_like(accn)
        accd[:, :] = jnp.zeros_like(accd)

        def _slab(g, carry):
            px = pxbf[:, pl.ds(_L * g, _L)]
            cw = cent[pl.ds(_NW * _RPS * g, _WIN), :]
            nss = -jnp.sum(cw * cw, axis=1, keepdims=True)
            ns_hi = nss.astype(jnp.bfloat16)
            ns_lo = (nss - ns_hi.astype(jnp.float32)).astype(jnp.bfloat16)
            cw_aug = jnp.concatenate(
                [(cw + cw).astype(jnp.bfloat16), ns_hi, ns_lo], axis=1)
            var = jnp.where(g == 0, 0, jnp.where(g == _NSLAB - 1, 2, 1))
            mask = mask_ref[pl.ds(var, 1), :, :][0].astype(jnp.float32)
            logits = jax.lax.dot_general(
                cw_aug, px, (((1,), (0,)), ((), ())),
                preferred_element_type=jnp.float32) + mask
            m = jnp.max(logits, axis=0, keepdims=True)
            e = jnp.exp(logits - m)
            zr = 1.0 / jnp.sum(e, axis=0, keepdims=True)
            a = e * zr
            contrib = jax.lax.dot_general(
                a.astype(jnp.bfloat16), px, (((1,), (1,)), ((), ())),
                preferred_element_type=jnp.float32)
            base = _NW * _RPS * g
            accn[pl.ds(base, _WIN), :] += contrib[:, :_C]
            accd[pl.ds(base, _WIN), :] += contrib[:, _C:_C + 1]

            @pl.when(s == _NINIT + _N_ITERS - 1)
            def _():
                wi = jax.lax.broadcasted_iota(jnp.int32, (_WIN, _L), 0)
                cand = jnp.where(logits >= m, wi, _WIN)
                lw = jnp.min(cand, axis=0)
                k = _NW * (_RPS * g - 1) + lw
                lab_ref[:, pl.ds(_L * g, _L)] = k.reshape(1, _L)

            return carry

        jax.lax.fori_loop(0, _NSLAB, _slab, 0)

    @pl.when(s == _NINIT + _N_ITERS - 1)
    def _():
        spf_ref[0, :, :] = accn[_NW:_NW + _K, :] / (accd[_NW:_NW + _K, :] +
                                                    1e-16)


def kernel(f):
    pix = f.reshape(_C, _H, _W)
    mask = jnp.asarray(_MASK_NP).astype(jnp.bfloat16)
    sel = jnp.asarray(_SEL_NP)
    spf, lab = pl.pallas_call(
        _ssn_body,
        grid=(_NINIT + _N_ITERS,),
        in_specs=[
            pl.BlockSpec(
                (_C, _IL // _W, _W),
                lambda s: (0, jnp.minimum(s, _NINIT - 1), 0)),
            pl.BlockSpec((3, _WIN, _L), lambda s: (0, 0, 0)),
            pl.BlockSpec((_RPS * _NW, _L), lambda s: (0, 0)),
        ],
        out_specs=[
            pl.BlockSpec((1, _K, _C), lambda s: (0, 0, 0)),
            pl.BlockSpec((1, _P), lambda s: (0, 0)),
        ],
        out_shape=[
            jax.ShapeDtypeStruct((1, _K, _C), jnp.float32),
            jax.ShapeDtypeStruct((1, _P), jnp.int32),
        ],
        scratch_shapes=[
            pltpu.VMEM(((_NH + 2) * _NW, _C), jnp.float32),
            pltpu.VMEM(((_NH + 2) * _NW, _C), jnp.float32),
            pltpu.VMEM(((_NH + 2) * _NW, 1), jnp.float32),
            pltpu.VMEM((_C + 2, _P), jnp.bfloat16),
        ],
    )(pix, mask, sel)
    return spf, lab
